# baseline (speedup 1.0000x reference)
import jax
import jax.numpy as jnp
from jax.experimental import pallas as pl
from jax.experimental.pallas import tpu as pltpu

_D0 = 3
_D1 = 5023
_D2 = 2048
_TOTAL = _D0 * _D1 * _D2
_BS = 64
_GRID = (_D1 + _BS - 1) // _BS
_TAIL_VALID = _D1 - (_GRID - 1) * _BS


def _mse_kernel(p_ref, r_ref, out_ref, acc_ref):
    i = pl.program_id(0)

    @pl.when(i == 0)
    def _init():
        acc_ref[...] = jnp.zeros_like(acc_ref)

    d = p_ref[...] - r_ref[...]
    dsq = d * d

    @pl.when(i < _GRID - 1)
    def _body():
        acc_ref[...] += jnp.sum(
            dsq.reshape(_D0 * _BS // 8, 8, _D2), axis=0)

    @pl.when(i == _GRID - 1)
    def _tail():
        row = jax.lax.broadcasted_iota(jnp.int32, (_D0, _BS, _D2), 1)
        masked = jnp.where(row < _TAIL_VALID, dsq, 0.0)
        acc = acc_ref[...] + jnp.sum(
            masked.reshape(_D0 * _BS // 8, 8, _D2), axis=0)
        out_ref[0] = jnp.sum(acc)


def kernel(pred_vertices, ref_vertices):
    p = jnp.transpose(pred_vertices, (2, 1, 0))
    r = jnp.transpose(ref_vertices, (2, 1, 0))
    total = pl.pallas_call(
        _mse_kernel,
        grid=(_GRID,),
        in_specs=[
            pl.BlockSpec((_D0, _BS, _D2), lambda i: (0, i, 0)),
            pl.BlockSpec((_D0, _BS, _D2), lambda i: (0, i, 0)),
        ],
        out_specs=pl.BlockSpec(memory_space=pltpu.MemorySpace.SMEM),
        out_shape=jax.ShapeDtypeStruct((1,), jnp.float32),
        scratch_shapes=[pltpu.VMEM((8, _D2), jnp.float32)],
    )(p, r)
    return (total[0] / _TOTAL).astype(jnp.float32)


# speedup vs baseline: 1.2517x; 1.2517x over previous
import jax
import jax.numpy as jnp
from jax.experimental import pallas as pl
from jax.experimental.pallas import tpu as pltpu

_D0 = 3
_D1 = 5023
_D2 = 2048
_HALF = _D2 // 2
_TOTAL = _D0 * _D1 * _D2
_BS = 128
_GRID = (_D1 + _BS - 1) // _BS
_TAIL_VALID = _D1 - (_GRID - 1) * _BS


def _psum(dsq):
    return jnp.sum(dsq.reshape(_D0 * _BS // 8, 8, _HALF), axis=0)


def _mse_kernel(p0_ref, p1_ref, r0_ref, r1_ref, out_ref, a0_ref, a1_ref):
    i = pl.program_id(0)

    @pl.when(i == 0)
    def _init():
        a0_ref[...] = jnp.zeros_like(a0_ref)
        a1_ref[...] = jnp.zeros_like(a1_ref)

    d0 = p0_ref[...] - r0_ref[...]
    d1 = p1_ref[...] - r1_ref[...]
    dsq0 = d0 * d0
    dsq1 = d1 * d1

    @pl.when(i < _GRID - 1)
    def _body():
        a0_ref[...] += _psum(dsq0)
        a1_ref[...] += _psum(dsq1)

    @pl.when(i == _GRID - 1)
    def _tail():
        row = jax.lax.broadcasted_iota(jnp.int32, (_D0, _BS, _HALF), 1)
        m0 = jnp.where(row < _TAIL_VALID, dsq0, 0.0)
        m1 = jnp.where(row < _TAIL_VALID, dsq1, 0.0)
        acc = a0_ref[...] + a1_ref[...] + _psum(m0) + _psum(m1)
        out_ref[0] = jnp.sum(acc)


def kernel(pred_vertices, ref_vertices):
    p = jnp.transpose(pred_vertices, (2, 1, 0))
    r = jnp.transpose(ref_vertices, (2, 1, 0))
    half_spec_lo = pl.BlockSpec((_D0, _BS, _HALF), lambda i: (0, i, 0))
    half_spec_hi = pl.BlockSpec((_D0, _BS, _HALF), lambda i: (0, i, 1))
    total = pl.pallas_call(
        _mse_kernel,
        grid=(_GRID,),
        in_specs=[half_spec_lo, half_spec_hi, half_spec_lo, half_spec_hi],
        out_specs=pl.BlockSpec(memory_space=pltpu.MemorySpace.SMEM),
        out_shape=jax.ShapeDtypeStruct((1,), jnp.float32),
        scratch_shapes=[pltpu.VMEM((8, _HALF), jnp.float32),
                        pltpu.VMEM((8, _HALF), jnp.float32)],
    )(p, p, r, r)
    return (total[0] / _TOTAL).astype(jnp.float32)
